# Initial kernel scaffold; baseline (speedup 1.0000x reference)
import functools

import jax
import jax.numpy as jnp
from jax import lax
from jax.experimental import pallas as pl
from jax.experimental.pallas import tpu as pltpu
from jax.experimental.pallas import tpu_sc as plsc

H = 16
L = 32
N = L * L


def _sc_body(b0_hbm, b1_hbm, out_hbm, b0_v, b1_v, grp_v, sem0, sem1):
    cid = lax.axis_index("c")
    sid = lax.axis_index("s")
    w = sid * 2 + cid

    pltpu.sync_copy(b0_hbm, b0_v)
    pltpu.sync_copy(b1_hbm, b1_v)

    lane = lax.iota(jnp.int32, 16)
    w_v = jnp.broadcast_to(w, (16,)).astype(jnp.int32)
    sems = (sem0, sem1)

    for h in range(H):
        buf = h % 2
        h_splat = jnp.full((16,), h, dtype=jnp.int32)

        s_vecs = [
            plsc.load_gather(b0_v, [h_splat, (32 + b) - w_v])
            for b in range(L)
        ]

        if h >= 2:
            pltpu.make_async_copy(
                grp_v.at[buf],
                out_hbm.at[h - 2, pl.ds(w * L, L)],
                sems[buf],
            ).wait()

        def build_row(c, carry):
            idx_lo = lane + (32 - c)
            row_lo = plsc.load_gather(b1_v, [h_splat, idx_lo])
            row_hi = plsc.load_gather(b1_v, [h_splat, idx_lo + 16])
            for b in range(L):
                grp_v[buf, c, pl.ds(b * L, 16)] = row_lo + s_vecs[b]
                grp_v[buf, c, pl.ds(b * L + 16, 16)] = row_hi + s_vecs[b]
            return carry

        lax.fori_loop(0, L, build_row, 0)

        pltpu.make_async_copy(
            grp_v.at[buf],
            out_hbm.at[h, pl.ds(w * L, L)],
            sems[buf],
        ).start()

    for h in (H - 2, H - 1):
        buf = h % 2
        pltpu.make_async_copy(
            grp_v.at[buf],
            out_hbm.at[h, pl.ds(w * L, L)],
            sems[buf],
        ).wait()


def kernel(bias_0, bias_1):
    mesh = plsc.VectorSubcoreMesh(core_axis_name="c", subcore_axis_name="s")
    run = functools.partial(
        pl.kernel,
        out_type=jax.ShapeDtypeStruct((H, N, N), jnp.float32),
        mesh=mesh,
        scratch_types=[
            pltpu.VMEM((H, 2 * L), jnp.float32),
            pltpu.VMEM((H, 2 * L), jnp.float32),
            pltpu.VMEM((2, L, N), jnp.float32),
            pltpu.SemaphoreType.DMA,
            pltpu.SemaphoreType.DMA,
        ],
    )(_sc_body)
    return run(bias_0, bias_1)


# speedup vs baseline: 9.2197x; 9.2197x over previous
import functools

import jax
import jax.numpy as jnp
from jax import lax
from jax.experimental import pallas as pl
from jax.experimental.pallas import tpu as pltpu
from jax.experimental.pallas import tpu_sc as plsc

H = 16
L = 32
N = L * L


def _sc_body(b0_hbm, b1_hbm, out_hbm, b0_v, b1_v, grp_v, sem0, sem1):
    cid = lax.axis_index("c")
    sid = lax.axis_index("s")
    w = sid * 2 + cid

    pltpu.sync_copy(b0_hbm, b0_v)
    pltpu.sync_copy(b1_hbm, b1_v)

    lane = lax.iota(jnp.int32, 16)
    w_v = jnp.broadcast_to(w, (16,)).astype(jnp.int32)
    sems = (sem0, sem1)

    for h in range(H):
        buf = h % 2
        h_base = h * 2 * L

        v_lo = b0_v[pl.ds(h_base + 32 - w, 16)]
        v_hi = b0_v[pl.ds(h_base + 48 - w, 16)]
        s_vecs = [
            jnp.take_along_axis(v_lo if b < 16 else v_hi,
                                jnp.full((16,), b % 16, dtype=jnp.int32),
                                axis=0)
            for b in range(L)
        ]

        if h >= 2:
            pltpu.make_async_copy(
                grp_v.at[buf],
                out_hbm.at[h - 2, pl.ds(w * L, L)],
                sems[buf],
            ).wait()

        def build_row(c, carry):
            off = h_base + 32 - c
            row_lo = b1_v[pl.ds(off, 16)]
            row_hi = b1_v[pl.ds(off + 16, 16)]
            for b in range(L):
                grp_v[buf, c, pl.ds(b * L, 16)] = row_lo + s_vecs[b]
                grp_v[buf, c, pl.ds(b * L + 16, 16)] = row_hi + s_vecs[b]
            return carry

        lax.fori_loop(0, L, build_row, 0)

        pltpu.make_async_copy(
            grp_v.at[buf],
            out_hbm.at[h, pl.ds(w * L, L)],
            sems[buf],
        ).start()

    for h in (H - 2, H - 1):
        buf = h % 2
        pltpu.make_async_copy(
            grp_v.at[buf],
            out_hbm.at[h, pl.ds(w * L, L)],
            sems[buf],
        ).wait()


def kernel(bias_0, bias_1):
    mesh = plsc.VectorSubcoreMesh(core_axis_name="c", subcore_axis_name="s")
    run = functools.partial(
        pl.kernel,
        out_type=jax.ShapeDtypeStruct((H, N, N), jnp.float32),
        mesh=mesh,
        scratch_types=[
            pltpu.VMEM((H * 2 * L,), jnp.float32),
            pltpu.VMEM((H * 2 * L,), jnp.float32),
            pltpu.VMEM((2, L, N), jnp.float32),
            pltpu.SemaphoreType.DMA,
            pltpu.SemaphoreType.DMA,
        ],
    )(_sc_body)
    return run(bias_0.reshape(-1), bias_1.reshape(-1))
